# baseline (device time: 34073 ns/iter reference)
import jax
import jax.numpy as jnp
from jax import lax
from jax.experimental import pallas as pl
from jax.experimental.pallas import tpu as pltpu

N_DEV = 4
B, SQ, D = 4, 256, 1024
HS = SQ // 2
HQL = 8
DH = 128
KVL = 2
NKV = KVL * DH
SCALE = 0.08838834764831843
BF = jnp.bfloat16
F32 = jnp.float32
I8 = jnp.int8


def kernel(x, Wq, Wo, Wk, Wv):
    my = lax.axis_index("i")
    Wk_loc = lax.dynamic_slice(Wk, (0, my * NKV), (D, NKV))
    Wv_loc = lax.dynamic_slice(Wv, (0, my * NKV), (D, NKV))

    def body(x_ref, wq_ref, wo_ref, wk_ref, wv_ref, out_ref,
             wqkv_bf, kv_cache, attn_ref, rs_sq, rs_ss, rs_rq, rs_rs,
             ag_sq, ag_ss, ag_rq, ag_rs,
             rs_send_sems, rs_recv_sems, ag_send_sems, ag_recv_sems):
        my_pos = lax.axis_index("i")
        peers = [(my_pos + 1 + j) % N_DEV for j in range(N_DEV - 1)]
        batches = peers + [my_pos]

        wqkv_bf[:, :D] = wq_ref[...].astype(BF)
        wqkv_bf[:, D:D + NKV] = wk_ref[...].astype(BF)
        wqkv_bf[:, D + NKV:] = wv_ref[...].astype(BF)
        wo = wo_ref[...].astype(BF)

        def compute_partial(j, e):
            b = batches[j]
            if e == 0:
                xf = x_ref[pl.ds(b, 1), :, :].reshape(SQ, D).astype(BF)
                kv_cache[j, :, :] = jnp.dot(
                    xf, wqkv_bf[:, D:], preferred_element_type=F32
                ).astype(BF)
            xh = x_ref[pl.ds(b, 1), pl.ds(e * HS, HS), :].reshape(
                HS, D).astype(BF)
            q = jnp.dot(xh, wqkv_bf[:, :D],
                        preferred_element_type=F32).astype(BF)
            for h in range(HQL):
                g = h // 4
                qh = q[:, h * DH:(h + 1) * DH]
                kh = kv_cache[j, :, g * DH:(g + 1) * DH]
                vh = kv_cache[j, :, NKV + g * DH:NKV + (g + 1) * DH]
                s = lax.dot_general(
                    qh, kh, (((1,), (1,)), ((), ())),
                    preferred_element_type=F32,
                ) * SCALE
                p = jnp.exp(s)
                l = jnp.sum(p, axis=-1, keepdims=True)
                oh = jnp.dot(p.astype(BF), vh, preferred_element_type=F32)
                attn_ref[:, h * DH:(h + 1) * DH] = (oh / l).astype(BF)
            return jnp.dot(attn_ref[...], wo, preferred_element_type=F32)

        def quantize(p):
            amax = jnp.maximum(
                jnp.max(jnp.abs(p), axis=0, keepdims=True), 1e-6)
            q = jnp.rint(p * (127.0 / amax)).astype(I8)
            return q, amax * (1.0 / 127.0)

        barrier_sem = pltpu.get_barrier_semaphore()
        for p in peers:
            pl.semaphore_signal(
                barrier_sem, inc=1,
                device_id=(p,), device_id_type=pl.DeviceIdType.MESH,
            )

        rdmas = []

        def start_pair(srcs, dsts, send_sems, recv_sems, e, jj, slot, tgt):
            for part, (src, dst) in enumerate(zip(srcs, dsts)):
                r = pltpu.make_async_remote_copy(
                    src_ref=src, dst_ref=dst,
                    send_sem=send_sems.at[e, jj, part],
                    recv_sem=recv_sems.at[e, slot, part],
                    device_id=(tgt,),
                    device_id_type=pl.DeviceIdType.MESH,
                )
                r.start()
                rdmas.append(r)

        def wait_pair(dsts, recv_sems, e, slot):
            for part, dst in enumerate(dsts):
                w = pltpu.make_async_remote_copy(
                    src_ref=dst, dst_ref=dst,
                    send_sem=recv_sems.at[e, slot, part],
                    recv_sem=recv_sems.at[e, slot, part],
                    device_id=(my_pos,),
                    device_id_type=pl.DeviceIdType.MESH,
                )
                w.wait_recv()

        def half_phase(e):
            for j in range(N_DEV - 1):
                q, s8 = quantize(compute_partial(j, e))
                rs_sq[e, j, :, :] = q
                rs_ss[e, j, :, :] = s8
                if e == 0 and j == 0:
                    pl.semaphore_wait(barrier_sem, N_DEV - 1)
                start_pair(
                    (rs_sq.at[e, j], rs_ss.at[e, j]),
                    (rs_rq.at[e, N_DEV - 2 - j], rs_rs.at[e, N_DEV - 2 - j]),
                    rs_send_sems, rs_recv_sems,
                    e, j, N_DEV - 2 - j, peers[j])

            acc = compute_partial(N_DEV - 1, e)
            for s in range(N_DEV - 1):
                wait_pair((rs_rq.at[e, s], rs_rs.at[e, s]),
                          rs_recv_sems, e, s)
                acc = acc + rs_rq[e, s, :, :].astype(F32) * rs_rs[e, s, :, :]

            out_ref[pl.ds(my_pos, 1), pl.ds(e * HS, HS), :] = (
                acc.astype(BF).reshape(1, HS, D))
            q, s8 = quantize(acc)
            ag_sq[e, :, :] = q
            ag_ss[e, :, :] = s8
            for j in range(N_DEV - 1):
                start_pair(
                    (ag_sq.at[e], ag_ss.at[e]),
                    (ag_rq.at[e, N_DEV - 2 - j], ag_rs.at[e, N_DEV - 2 - j]),
                    ag_send_sems, ag_recv_sems,
                    e, j, N_DEV - 2 - j, peers[j])

        half_phase(0)
        half_phase(1)

        for e in range(2):
            for s in range(N_DEV - 1):
                wait_pair((ag_rq.at[e, s], ag_rs.at[e, s]),
                          ag_recv_sems, e, s)
                c = (my_pos + 1 + s) % N_DEV
                out_ref[pl.ds(c, 1), pl.ds(e * HS, HS), :] = (
                    (ag_rq[e, s, :, :].astype(F32) * ag_rs[e, s, :, :])
                    .astype(BF).reshape(1, HS, D))

        for r in rdmas:
            r.wait_send()

    return pl.pallas_call(
        body,
        out_shape=jax.ShapeDtypeStruct((B, SQ, D), BF),
        in_specs=[pl.BlockSpec(memory_space=pltpu.VMEM)] * 5,
        out_specs=pl.BlockSpec(memory_space=pltpu.VMEM),
        scratch_shapes=[
            pltpu.VMEM((D, D + 2 * NKV), BF),
            pltpu.VMEM((N_DEV, SQ, 2 * NKV), BF),
            pltpu.VMEM((HS, D), BF),
            pltpu.VMEM((2, N_DEV - 1, HS, D), I8),
            pltpu.VMEM((2, N_DEV - 1, 1, D), F32),
            pltpu.VMEM((2, N_DEV - 1, HS, D), I8),
            pltpu.VMEM((2, N_DEV - 1, 1, D), F32),
            pltpu.VMEM((2, HS, D), I8),
            pltpu.VMEM((2, 1, D), F32),
            pltpu.VMEM((2, N_DEV - 1, HS, D), I8),
            pltpu.VMEM((2, N_DEV - 1, 1, D), F32),
            pltpu.SemaphoreType.DMA((2, N_DEV - 1, 2)),
            pltpu.SemaphoreType.DMA((2, N_DEV - 1, 2)),
            pltpu.SemaphoreType.DMA((2, N_DEV - 1, 2)),
            pltpu.SemaphoreType.DMA((2, N_DEV - 1, 2)),
        ],
        compiler_params=pltpu.CompilerParams(collective_id=0),
    )(x, Wq, Wo, Wk_loc, Wv_loc)


# device time: 32354 ns/iter; 1.0531x vs baseline; 1.0531x over previous
import jax
import jax.numpy as jnp
from jax import lax
from jax.experimental import pallas as pl
from jax.experimental.pallas import tpu as pltpu

N_DEV = 4
B, SQ, D = 4, 256, 1024
HQL = 8
DH = 128
KVL = 2
NKV = KVL * DH
SCALE = 0.08838834764831843
BF = jnp.bfloat16
F32 = jnp.float32
I8 = jnp.int8


def kernel(x, Wq, Wo, Wk, Wv):
    my = lax.axis_index("i")
    Wk_loc = lax.dynamic_slice(Wk, (0, my * NKV), (D, NKV))
    Wv_loc = lax.dynamic_slice(Wv, (0, my * NKV), (D, NKV))

    def body(x_ref, wq_ref, wo_ref, wk_ref, wv_ref, out_ref,
             wqkv_bf, attn_ref, rs_sq, rs_ss, rs_rq, rs_rs,
             ag_sq, ag_ss, ag_rq, ag_rs,
             rs_send_sems, rs_recv_sems, ag_send_sems, ag_recv_sems):
        my_pos = lax.axis_index("i")
        peers = [(my_pos + 1 + j) % N_DEV for j in range(N_DEV - 1)]

        wqkv_bf[:, :D] = wq_ref[...].astype(BF)
        wqkv_bf[:, D:D + NKV] = wk_ref[...].astype(BF)
        wqkv_bf[:, D + NKV:] = wv_ref[...].astype(BF)
        wo = wo_ref[...].astype(BF)

        def compute_partial(b):
            xb = x_ref[pl.ds(b, 1), :, :].reshape(SQ, D).astype(BF)
            qkv = jnp.dot(xb, wqkv_bf[...],
                          preferred_element_type=F32).astype(BF)
            for h in range(HQL):
                g = h // 4
                qh = qkv[:, h * DH:(h + 1) * DH]
                kh = qkv[:, D + g * DH:D + (g + 1) * DH]
                vh = qkv[:, D + NKV + g * DH:D + NKV + (g + 1) * DH]
                s = lax.dot_general(
                    qh, kh, (((1,), (1,)), ((), ())),
                    preferred_element_type=F32,
                ) * SCALE
                p = jnp.exp(s)
                l = jnp.sum(p, axis=-1, keepdims=True)
                oh = jnp.dot(p.astype(BF), vh, preferred_element_type=F32)
                attn_ref[:, h * DH:(h + 1) * DH] = (oh / l).astype(BF)
            return jnp.dot(attn_ref[...], wo, preferred_element_type=F32)

        def quantize(p):
            amax = jnp.maximum(
                jnp.max(jnp.abs(p), axis=0, keepdims=True), 1e-6)
            q = jnp.rint(p * (127.0 / amax)).astype(I8)
            return q, amax * (1.0 / 127.0)

        barrier_sem = pltpu.get_barrier_semaphore()
        for p in peers:
            pl.semaphore_signal(
                barrier_sem, inc=1,
                device_id=(p,), device_id_type=pl.DeviceIdType.MESH,
            )

        rdmas = []

        def start_pair(sq, ss, dq, ds, send_sems, recv_sems, jj, slot, tgt):
            for part, (src, dst) in enumerate(((sq, dq), (ss, ds))):
                r = pltpu.make_async_remote_copy(
                    src_ref=src, dst_ref=dst,
                    send_sem=send_sems.at[jj, part],
                    recv_sem=recv_sems.at[slot, part],
                    device_id=(tgt,),
                    device_id_type=pl.DeviceIdType.MESH,
                )
                r.start()
                rdmas.append(r)

        def wait_pair(dq, ds, recv_sems, slot):
            for part, dst in enumerate((dq, ds)):
                w = pltpu.make_async_remote_copy(
                    src_ref=dst, dst_ref=dst,
                    send_sem=recv_sems.at[slot, part],
                    recv_sem=recv_sems.at[slot, part],
                    device_id=(my_pos,),
                    device_id_type=pl.DeviceIdType.MESH,
                )
                w.wait_recv()

        for j in range(N_DEV - 1):
            q, s8 = quantize(compute_partial(peers[j]))
            rs_sq[j, :, :] = q
            rs_ss[j, :, :] = s8
            if j == 0:
                pl.semaphore_wait(barrier_sem, N_DEV - 1)
            start_pair(rs_sq.at[j], rs_ss.at[j],
                       rs_rq.at[N_DEV - 2 - j], rs_rs.at[N_DEV - 2 - j],
                       rs_send_sems, rs_recv_sems,
                       j, N_DEV - 2 - j, peers[j])

        local = compute_partial(my_pos)

        acc = local
        for s in reversed(range(N_DEV - 1)):
            wait_pair(rs_rq.at[s], rs_rs.at[s], rs_recv_sems, s)
            acc = acc + rs_rq[s, :, :].astype(F32) * rs_rs[s, :, :]

        out_ref[pl.ds(my_pos, 1), :, :] = acc.astype(BF).reshape(1, SQ, D)
        q, s8 = quantize(acc)
        ag_sq[0, :, :] = q
        ag_ss[0, :, :] = s8

        for j in range(N_DEV - 1):
            start_pair(ag_sq.at[0], ag_ss.at[0],
                       ag_rq.at[N_DEV - 2 - j], ag_rs.at[N_DEV - 2 - j],
                       ag_send_sems, ag_recv_sems,
                       j, N_DEV - 2 - j, peers[j])
        for s in range(N_DEV - 1):
            wait_pair(ag_rq.at[s], ag_rs.at[s], ag_recv_sems, s)
            c = (my_pos + 1 + s) % N_DEV
            out_ref[pl.ds(c, 1), :, :] = (
                (ag_rq[s, :, :].astype(F32) * ag_rs[s, :, :])
                .astype(BF).reshape(1, SQ, D))

        for r in rdmas:
            r.wait_send()

    return pl.pallas_call(
        body,
        out_shape=jax.ShapeDtypeStruct((B, SQ, D), BF),
        in_specs=[pl.BlockSpec(memory_space=pltpu.VMEM)] * 5,
        out_specs=pl.BlockSpec(memory_space=pltpu.VMEM),
        scratch_shapes=[
            pltpu.VMEM((D, D + 2 * NKV), BF),
            pltpu.VMEM((SQ, D), BF),
            pltpu.VMEM((N_DEV - 1, SQ, D), I8),
            pltpu.VMEM((N_DEV - 1, 1, D), F32),
            pltpu.VMEM((N_DEV - 1, SQ, D), I8),
            pltpu.VMEM((N_DEV - 1, 1, D), F32),
            pltpu.VMEM((1, SQ, D), I8),
            pltpu.VMEM((1, 1, D), F32),
            pltpu.VMEM((N_DEV - 1, SQ, D), I8),
            pltpu.VMEM((N_DEV - 1, 1, D), F32),
            pltpu.SemaphoreType.DMA((N_DEV - 1, 2)),
            pltpu.SemaphoreType.DMA((N_DEV - 1, 2)),
            pltpu.SemaphoreType.DMA((N_DEV - 1, 2)),
            pltpu.SemaphoreType.DMA((N_DEV - 1, 2)),
        ],
        compiler_params=pltpu.CompilerParams(collective_id=0),
    )(x, Wq, Wo, Wk_loc, Wv_loc)
